# Initial kernel scaffold; baseline (speedup 1.0000x reference)
import jax
import jax.numpy as jnp
from jax.experimental import pallas as pl


def kernel(cell_attr, edge_attr, node_embedding, edge_index, face, W, b):
    raise NotImplementedError("write your pallas kernel here")


# speedup vs baseline: 15.7855x; 15.7855x over previous
import math

import jax
import jax.numpy as jnp
from jax import lax
from jax.experimental import pallas as pl
from jax.experimental.pallas import tpu as pltpu
from jax.experimental.pallas import tpu_sc as plsc

_D = 128
_NN = 10000
_NE = 320000
_NCELL = 20000
_L = 16
_NCORES = 2
_NSUB = 16
_EC = 40
_CC = 32
_NCH_E = _NE // _EC
_TPW_E = _NCH_E // (_NCORES * _NSUB)
_CPAD = 20480
_NCH_C = _CPAD // _CC
_TPW_C = _NCH_C // (_NCORES * _NSUB)
_NPAD = 10240
_ROWS = _NPAD // _NSUB


def _tree_sum(vs):
    while len(vs) > 1:
        nxt = [a + b for a, b in zip(vs[::2], vs[1::2])]
        if len(vs) % 2:
            nxt[-1] = nxt[-1] + vs[-1]
        vs = nxt
    return vs[0]


def _edge_stage(recv, send, eattr, nemb, acc_out, den_out,
                idx_r, idx_s, ebuf, nbr, nbs, exm_r, exm_s,
                den_sh, acc_sh, sem_r, sem_s):
    c = lax.axis_index("c")
    s = lax.axis_index("s")
    zv = jnp.zeros((_L,), jnp.float32)

    def zero_row(i, carry):
        for k in range(_D // _L):
            ebuf[i, pl.ds(k * _L, _L)] = zv
        exm_r[i, :] = zv
        return carry

    lax.fori_loop(0, _EC, zero_row, 0)
    row0 = s * _ROWS
    for j in range(_ROWS // _EC):
        pltpu.sync_copy(ebuf, acc_sh.at[pl.ds(row0 + j * _EC, _EC)])
        pltpu.sync_copy(exm_r, den_sh.at[pl.ds(row0 + j * _EC, _EC)])
    plsc.subcore_barrier()

    inv = jnp.float32(1.0 / math.sqrt(float(_D)))

    def chunk_body(t, carry):
        chunk = (t * _NSUB + s) * _NCORES + c
        base = chunk * _EC
        pltpu.sync_copy(recv.at[pl.ds(base, _EC)], idx_r)
        pltpu.sync_copy(send.at[pl.ds(base, _EC)], idx_s)
        cp_r = pltpu.async_copy(nemb.at[idx_r], nbr, sem_r)
        cp_s = pltpu.async_copy(nemb.at[idx_s], nbs, sem_s)
        pltpu.sync_copy(eattr.at[pl.ds(base, _EC)], ebuf)
        cp_r.wait()
        cp_s.wait()

        def edge_body(i, carry2):
            pr = []
            ps = []
            for k in range(_D // _L):
                e = ebuf[i, pl.ds(k * _L, _L)]
                pr.append(e * nbr[i, pl.ds(k * _L, _L)])
                ps.append(e * nbs[i, pl.ds(k * _L, _L)])
            sr = jnp.sum(_tree_sum(pr)) * inv
            ss = jnp.sum(_tree_sum(ps)) * inv
            exr = jnp.exp(jnp.broadcast_to(sr, (_L,)))
            exs = jnp.exp(jnp.broadcast_to(ss, (_L,)))
            exm_r[i, :] = exr
            exm_s[i, :] = exs
            for k in range(_D // _L):
                e = ebuf[i, pl.ds(k * _L, _L)]
                nbr[i, pl.ds(k * _L, _L)] = e * exr
                nbs[i, pl.ds(k * _L, _L)] = e * exs
            return carry2

        lax.fori_loop(0, _EC, edge_body, 0)
        pltpu.sync_copy(nbr, acc_sh.at[idx_r], add=True)
        pltpu.sync_copy(nbs, acc_sh.at[idx_s], add=True)
        pltpu.sync_copy(exm_r, den_sh.at[idx_r], add=True)
        pltpu.sync_copy(exm_s, den_sh.at[idx_s], add=True)
        return carry

    lax.fori_loop(0, _TPW_E, chunk_body, 0)
    plsc.subcore_barrier()
    out0 = c * _NPAD + row0
    pltpu.sync_copy(acc_sh.at[pl.ds(row0, _ROWS)], acc_out.at[pl.ds(out0, _ROWS)])
    pltpu.sync_copy(den_sh.at[pl.ds(row0, _ROWS)], den_out.at[pl.ds(out0, _ROWS)])


_edge_call = pl.kernel(
    _edge_stage,
    out_type=[
        jax.ShapeDtypeStruct((_NCORES * _NPAD, _D), jnp.float32),
        jax.ShapeDtypeStruct((_NCORES * _NPAD, _L), jnp.float32),
    ],
    mesh=plsc.VectorSubcoreMesh(core_axis_name="c", subcore_axis_name="s"),
    compiler_params=pltpu.CompilerParams(needs_layout_passes=False),
    scratch_types=[
        pltpu.VMEM((_EC,), jnp.int32),
        pltpu.VMEM((_EC,), jnp.int32),
        pltpu.VMEM((_EC, _D), jnp.float32),
        pltpu.VMEM((_EC, _D), jnp.float32),
        pltpu.VMEM((_EC, _D), jnp.float32),
        pltpu.VMEM((_EC, _L), jnp.float32),
        pltpu.VMEM((_EC, _L), jnp.float32),
        pltpu.VMEM_SHARED((_NPAD, _L), jnp.float32),
        pltpu.VMEM_SHARED((_NPAD, _D), jnp.float32),
        pltpu.SemaphoreType.DMA,
        pltpu.SemaphoreType.DMA,
    ],
)


def _proj_body(acc0, acc1, den0, den1, w, o):
    d = den0[:, 0:1] + den1[:, 0:1] + 1e-16
    agg = (acc0[...] + acc1[...]) / d
    o[...] = jnp.dot(agg, w[...], preferred_element_type=jnp.float32)


_proj_call = pl.pallas_call(
    _proj_body,
    grid=(10,),
    in_specs=[
        pl.BlockSpec((_NN // 10, _D), lambda i: (i, 0)),
        pl.BlockSpec((_NN // 10, _D), lambda i: (i, 0)),
        pl.BlockSpec((_NN // 10, _L), lambda i: (i, 0)),
        pl.BlockSpec((_NN // 10, _L), lambda i: (i, 0)),
        pl.BlockSpec((_D, _D), lambda i: (0, 0)),
    ],
    out_specs=pl.BlockSpec((_NN // 10, _D), lambda i: (i, 0)),
    out_shape=jax.ShapeDtypeStruct((_NN, _D), jnp.float32),
)


def _ctop_body(x, w, bb, o):
    o[...] = jnp.dot(x[...], w[...], preferred_element_type=jnp.float32) + bb[...]


_ctop_call = pl.pallas_call(
    _ctop_body,
    grid=(10,),
    in_specs=[
        pl.BlockSpec((_CPAD // 10, _D), lambda i: (i, 0)),
        pl.BlockSpec((_D, _D), lambda i: (0, 0)),
        pl.BlockSpec((1, _D), lambda i: (0, 0)),
    ],
    out_specs=pl.BlockSpec((_CPAD // 10, _D), lambda i: (i, 0)),
    out_shape=jax.ShapeDtypeStruct((_CPAD, _D), jnp.float32),
)


def _cell_stage(f0h, f1h, f2h, ctoph, nprojh, cell_out, nsum_out, cnt_out,
                i0, i1, i2, p0, p1, p2, tb, ones,
                cnt_sh, nsum_sh, sem0, sem1, sem2):
    c = lax.axis_index("c")
    s = lax.axis_index("s")
    zv = jnp.zeros((_L,), jnp.float32)

    def zero_row(i, carry):
        for k in range(_D // _L):
            tb[i, pl.ds(k * _L, _L)] = zv
        ones[i, :] = zv
        return carry

    lax.fori_loop(0, _CC, zero_row, 0)
    row0 = s * _ROWS
    for j in range(_ROWS // _CC):
        pltpu.sync_copy(tb, nsum_sh.at[pl.ds(row0 + j * _CC, _CC)])
        pltpu.sync_copy(ones, cnt_sh.at[pl.ds(row0 + j * _CC, _CC)])
    ov = jnp.ones((_L,), jnp.float32)

    def ones_row(i, carry):
        ones[i, :] = ov
        return carry

    lax.fori_loop(0, _CC, ones_row, 0)
    plsc.subcore_barrier()

    third = jnp.float32(1.0 / 3.0)

    def chunk_body(t, carry):
        chunk = (t * _NSUB + s) * _NCORES + c
        base = chunk * _CC
        pltpu.sync_copy(f0h.at[pl.ds(base, _CC)], i0)
        pltpu.sync_copy(f1h.at[pl.ds(base, _CC)], i1)
        pltpu.sync_copy(f2h.at[pl.ds(base, _CC)], i2)
        cp0 = pltpu.async_copy(nprojh.at[i0], p0, sem0)
        cp1 = pltpu.async_copy(nprojh.at[i1], p1, sem1)
        cp2 = pltpu.async_copy(nprojh.at[i2], p2, sem2)
        pltpu.sync_copy(ctoph.at[pl.ds(base, _CC)], tb)
        cp0.wait()
        cp1.wait()
        cp2.wait()

        def cell_body(i, carry2):
            for k in range(_D // _L):
                sl = pl.ds(k * _L, _L)
                p0[i, sl] = (p0[i, sl] + p1[i, sl] + p2[i, sl]) * third + tb[i, sl]
            return carry2

        lax.fori_loop(0, _CC, cell_body, 0)
        pltpu.sync_copy(p0, cell_out.at[pl.ds(base, _CC)])
        pltpu.sync_copy(p0, nsum_sh.at[i0], add=True)
        pltpu.sync_copy(p0, nsum_sh.at[i1], add=True)
        pltpu.sync_copy(p0, nsum_sh.at[i2], add=True)
        pltpu.sync_copy(ones, cnt_sh.at[i0], add=True)
        pltpu.sync_copy(ones, cnt_sh.at[i1], add=True)
        pltpu.sync_copy(ones, cnt_sh.at[i2], add=True)
        return carry

    lax.fori_loop(0, _TPW_C, chunk_body, 0)
    plsc.subcore_barrier()
    out0 = c * _NPAD + row0
    pltpu.sync_copy(nsum_sh.at[pl.ds(row0, _ROWS)], nsum_out.at[pl.ds(out0, _ROWS)])
    pltpu.sync_copy(cnt_sh.at[pl.ds(row0, _ROWS)], cnt_out.at[pl.ds(out0, _ROWS)])


_cell_call = pl.kernel(
    _cell_stage,
    out_type=[
        jax.ShapeDtypeStruct((_CPAD, _D), jnp.float32),
        jax.ShapeDtypeStruct((_NCORES * _NPAD, _D), jnp.float32),
        jax.ShapeDtypeStruct((_NCORES * _NPAD, _L), jnp.float32),
    ],
    mesh=plsc.VectorSubcoreMesh(core_axis_name="c", subcore_axis_name="s"),
    compiler_params=pltpu.CompilerParams(needs_layout_passes=False),
    scratch_types=[
        pltpu.VMEM((_CC,), jnp.int32),
        pltpu.VMEM((_CC,), jnp.int32),
        pltpu.VMEM((_CC,), jnp.int32),
        pltpu.VMEM((_CC, _D), jnp.float32),
        pltpu.VMEM((_CC, _D), jnp.float32),
        pltpu.VMEM((_CC, _D), jnp.float32),
        pltpu.VMEM((_CC, _D), jnp.float32),
        pltpu.VMEM((_CC, _L), jnp.float32),
        pltpu.VMEM_SHARED((_NPAD, _L), jnp.float32),
        pltpu.VMEM_SHARED((_NPAD, _D), jnp.float32),
        pltpu.SemaphoreType.DMA,
        pltpu.SemaphoreType.DMA,
        pltpu.SemaphoreType.DMA,
    ],
)


def _final_body(s0, s1, c0, c1, o):
    cnt = jnp.clip(c0[:, 0:1] + c1[:, 0:1], 1.0, None)
    o[...] = (s0[...] + s1[...]) / cnt


_final_call = pl.pallas_call(
    _final_body,
    grid=(10,),
    in_specs=[
        pl.BlockSpec((_NN // 10, _D), lambda i: (i, 0)),
        pl.BlockSpec((_NN // 10, _D), lambda i: (i, 0)),
        pl.BlockSpec((_NN // 10, _L), lambda i: (i, 0)),
        pl.BlockSpec((_NN // 10, _L), lambda i: (i, 0)),
    ],
    out_specs=pl.BlockSpec((_NN // 10, _D), lambda i: (i, 0)),
    out_shape=jax.ShapeDtypeStruct((_NN, _D), jnp.float32),
)


def kernel(cell_attr, edge_attr, node_embedding, edge_index, face, W, b):
    send = edge_index[0]
    recv = edge_index[1]
    acc, den = _edge_call(recv, send, edge_attr, node_embedding)
    node_proj = _proj_call(acc[:_NN], acc[_NN:], den[:_NN], den[_NN:], W[_D:])
    node_proj_p = jnp.pad(node_proj, ((0, _L), (0, 0)))
    cell_attr_p = jnp.pad(cell_attr, ((0, _CPAD - _NCELL), (0, 0)))
    cell_top = _ctop_call(cell_attr_p, W[:_D], b.reshape(1, _D))
    fpad = jnp.pad(face, ((0, 0), (0, _CPAD - _NCELL)), constant_values=_NN)
    cell_new_p, nsum, cnt = _cell_call(fpad[0], fpad[1], fpad[2], cell_top, node_proj_p)
    node_attr = _final_call(nsum[:_NN], nsum[_NPAD:_NPAD + _NN],
                            cnt[:_NN], cnt[_NPAD:_NPAD + _NN])
    return cell_new_p[:_NCELL], node_attr
